# Initial kernel scaffold; baseline (speedup 1.0000x reference)
import jax
import jax.numpy as jnp
from jax.experimental import pallas as pl


def kernel(deformed_vertices, face, face_ids, face_bary):
    raise NotImplementedError("write your pallas kernel here")


# speedup vs baseline: 9.8162x; 9.8162x over previous
import functools

import jax
import jax.numpy as jnp
from jax import lax
from jax.experimental import pallas as pl
from jax.experimental.pallas import tpu as pltpu
from jax.experimental.pallas import tpu_sc as plsc

_N_GAUSS = 2_000_000
_B = 128
_NCHUNKS = _N_GAUSS // _B
_NC = 2
_NS = 16
_NW = _NC * _NS
_L = 16


def _body(fid_hbm, f0_hbm, f1_hbm, f2_hbm, vx_hbm, vy_hbm, vz_hbm, bary_hbm,
          out_hbm,
          fid_v, vid0_v, vid1_v, vid2_v, g_v, bary_v, out_v,
          sem_a, sem_b):
    wid = lax.axis_index("s") * _NC + lax.axis_index("c")
    n_w = (_NCHUNKS - wid + _NW - 1) // _NW
    lane = lax.iota(jnp.int32, _L)
    planes = (vx_hbm, vy_hbm, vz_hbm)

    def chunk_body(ci, carry):
        chunk = wid + ci * _NW
        base = pl.multiple_of(chunk * _B, _B)
        base3 = pl.multiple_of(chunk * (3 * _B), 3 * _B)

        cp_bary = pltpu.async_copy(bary_hbm.at[pl.ds(base3, 3 * _B)], bary_v,
                                   sem_b)
        pltpu.sync_copy(fid_hbm.at[pl.ds(base, _B)], fid_v)

        cps = [pltpu.async_copy(f_hbm.at[fid_v], vid_v, sem_a)
               for f_hbm, vid_v in
               ((f0_hbm, vid0_v), (f1_hbm, vid1_v), (f2_hbm, vid2_v))]
        for cp in cps:
            cp.wait()

        crs = []
        for j, vid_v in enumerate((vid0_v, vid1_v, vid2_v)):
            for c in range(3):
                crs.append(pltpu.async_copy(planes[c].at[vid_v],
                                            g_v.at[3 * j + c], sem_a))
        for cr in crs:
            cr.wait()
        cp_bary.wait()

        for t in range(_B // _L):
            sl = pl.ds(t * _L, _L)
            idx3 = (lane + (t * _L)) * 3
            w0 = plsc.load_gather(bary_v, [idx3])
            w1 = plsc.load_gather(bary_v, [idx3 + 1])
            w2 = plsc.load_gather(bary_v, [idx3 + 2])
            for c in range(3):
                acc = (w0 * g_v[0 + c, sl] + w1 * g_v[3 + c, sl]
                       + w2 * g_v[6 + c, sl])
                plsc.store_scatter(out_v, [idx3 + c], acc)

        pltpu.sync_copy(out_v, out_hbm.at[pl.ds(base3, 3 * _B)])
        return carry

    lax.fori_loop(0, n_w, chunk_body, 0)


@jax.jit
def _sc_interp(fid, f0, f1, f2, vx, vy, vz, bary_flat):
    mesh = plsc.VectorSubcoreMesh(core_axis_name="c", subcore_axis_name="s")
    run = functools.partial(
        pl.kernel,
        mesh=mesh,
        compiler_params=pltpu.CompilerParams(needs_layout_passes=False),
        out_type=jax.ShapeDtypeStruct((3 * _N_GAUSS,), jnp.float32),
        scratch_types=[
            pltpu.VMEM((_B,), jnp.int32),
            pltpu.VMEM((_B,), jnp.int32),
            pltpu.VMEM((_B,), jnp.int32),
            pltpu.VMEM((_B,), jnp.int32),
            pltpu.VMEM((9, _B), jnp.float32),
            pltpu.VMEM((3 * _B,), jnp.float32),
            pltpu.VMEM((3 * _B,), jnp.float32),
            pltpu.SemaphoreType.DMA,
            pltpu.SemaphoreType.DMA,
        ],
    )(_body)
    return run(fid, f0, f1, f2, vx, vy, vz, bary_flat)


def kernel(deformed_vertices, face, face_ids, face_bary):
    f0 = face[0]
    f1 = face[1]
    f2 = face[2]
    vx = deformed_vertices[:, 0]
    vy = deformed_vertices[:, 1]
    vz = deformed_vertices[:, 2]
    bary_flat = face_bary.reshape(-1)
    out_flat = _sc_interp(face_ids, f0, f1, f2, vx, vy, vz, bary_flat)
    return out_flat.reshape(_N_GAUSS, 3)


# speedup vs baseline: 10.8214x; 1.1024x over previous
import functools

import jax
import jax.numpy as jnp
from jax import lax
from jax.experimental import pallas as pl
from jax.experimental.pallas import tpu as pltpu
from jax.experimental.pallas import tpu_sc as plsc

_N_GAUSS = 2_000_000
_B = 128
_NCHUNKS = _N_GAUSS // _B
_NC = 2
_NS = 16
_NW = _NC * _NS
_L = 16


def _body(fid_hbm, f0_hbm, f1_hbm, f2_hbm, vx_hbm, vy_hbm, vz_hbm, bary_hbm,
          out_hbm,
          fid_v, vid_v, g_v, bary_v, out_v,
          s_fid, s_bary, s_vid, s_pl, s_out):
    wid = lax.axis_index("s") * _NC + lax.axis_index("c")
    n_w = (_NCHUNKS - wid + _NW - 1) // _NW
    lane = lax.iota(jnp.int32, _L)
    planes = (vx_hbm, vy_hbm, vz_hbm)
    faces = (f0_hbm, f1_hbm, f2_hbm)

    def chunk(i):
        return wid + i * _NW

    def fid_copy(i):
        base = pl.multiple_of(chunk(i) * _B, _B)
        return pltpu.make_async_copy(
            fid_hbm.at[pl.ds(base, _B)], fid_v.at[i % 2], s_fid)

    def bary_copy(i):
        base3 = pl.multiple_of(chunk(i) * (3 * _B), 3 * _B)
        return pltpu.make_async_copy(
            bary_hbm.at[pl.ds(base3, 3 * _B)], bary_v.at[i % 2], s_bary)

    def vid_copies(i):
        return [pltpu.make_async_copy(
                    faces[j].at[fid_v.at[i % 2]], vid_v.at[i % 3, j], s_vid)
                for j in range(3)]

    def plane_copies(i):
        return [pltpu.make_async_copy(
                    planes[c].at[vid_v.at[i % 3, j]],
                    g_v.at[(i % 2) * 9 + 3 * j + c], s_pl)
                for j in range(3) for c in range(3)]

    def out_copy(i):
        base3 = pl.multiple_of(chunk(i) * (3 * _B), 3 * _B)
        return pltpu.make_async_copy(
            out_v.at[i % 2], out_hbm.at[pl.ds(base3, 3 * _B)], s_out)

    def start(cps):
        if not isinstance(cps, (list, tuple)):
            cps = [cps]
        for cp in cps:
            cp.start()

    def wait(cps):
        if not isinstance(cps, (list, tuple)):
            cps = [cps]
        for cp in cps:
            cp.wait()

    start(fid_copy(0))
    start(fid_copy(1))
    start(bary_copy(0))
    wait(fid_copy(0))
    start(vid_copies(0))
    wait(vid_copies(0))
    start(plane_copies(0))
    wait(fid_copy(1))
    start(vid_copies(1))
    start(fid_copy(2))

    def loop_body(i, carry):
        p = i % 2

        @pl.when(i + 1 < n_w)
        def _():
            wait(vid_copies(i + 1))
            start(plane_copies(i + 1))
            start(bary_copy(i + 1))

        @pl.when(i + 2 < n_w)
        def _():
            wait(fid_copy(i + 2))
            start(vid_copies(i + 2))

        @pl.when(i + 3 < n_w)
        def _():
            start(fid_copy(i + 3))

        wait(plane_copies(i))
        wait(bary_copy(i))

        @pl.when(i >= 2)
        def _():
            wait(out_copy(i - 2))

        pvec = jnp.full((_L,), p, jnp.int32)
        gbase = p * 9
        for t in range(_B // _L):
            idx = lane + (t * _L)
            idx3 = idx * 3
            w0 = plsc.load_gather(bary_v, [pvec, idx3])
            w1 = plsc.load_gather(bary_v, [pvec, idx3 + 1])
            w2 = plsc.load_gather(bary_v, [pvec, idx3 + 2])
            for c in range(3):
                g0 = plsc.load_gather(g_v, [jnp.full((_L,), gbase + c,
                                                     jnp.int32), idx])
                g1 = plsc.load_gather(g_v, [jnp.full((_L,), gbase + 3 + c,
                                                     jnp.int32), idx])
                g2 = plsc.load_gather(g_v, [jnp.full((_L,), gbase + 6 + c,
                                                     jnp.int32), idx])
                acc = w0 * g0 + w1 * g1 + w2 * g2
                plsc.store_scatter(out_v, [pvec, idx3 + c], acc)
        start(out_copy(i))
        return carry

    lax.fori_loop(0, n_w, loop_body, 0)
    wait(out_copy(n_w - 2))
    wait(out_copy(n_w - 1))


@jax.jit
def _sc_interp(fid, f0, f1, f2, vx, vy, vz, bary_flat):
    mesh = plsc.VectorSubcoreMesh(core_axis_name="c", subcore_axis_name="s")
    run = functools.partial(
        pl.kernel,
        mesh=mesh,
        compiler_params=pltpu.CompilerParams(needs_layout_passes=False),
        out_type=jax.ShapeDtypeStruct((3 * _N_GAUSS,), jnp.float32),
        scratch_types=[
            pltpu.VMEM((2, _B), jnp.int32),
            pltpu.VMEM((3, 3, _B), jnp.int32),
            pltpu.VMEM((18, _B), jnp.float32),
            pltpu.VMEM((2, 3 * _B), jnp.float32),
            pltpu.VMEM((2, 3 * _B), jnp.float32),
            pltpu.SemaphoreType.DMA,
            pltpu.SemaphoreType.DMA,
            pltpu.SemaphoreType.DMA,
            pltpu.SemaphoreType.DMA,
            pltpu.SemaphoreType.DMA,
        ],
    )(_body)
    return run(fid, f0, f1, f2, vx, vy, vz, bary_flat)


def kernel(deformed_vertices, face, face_ids, face_bary):
    f0 = face[0]
    f1 = face[1]
    f2 = face[2]
    vx = deformed_vertices[:, 0]
    vy = deformed_vertices[:, 1]
    vz = deformed_vertices[:, 2]
    bary_flat = face_bary.reshape(-1)
    out_flat = _sc_interp(face_ids, f0, f1, f2, vx, vy, vz, bary_flat)
    return out_flat.reshape(_N_GAUSS, 3)
